# Initial kernel scaffold; baseline (speedup 1.0000x reference)
import jax
import jax.numpy as jnp
from jax import lax
from jax.experimental import pallas as pl
from jax.experimental.pallas import tpu as pltpu
from jax.experimental.pallas import tpu_sc as plsc

N = 10000
D = 128
E = 320000
NC = 2
NS = 16
NW = NC * NS
PT = E // NW
K = 80
NB = PT // K
RP = N // NS
NP = 10240
RPP = NP // NS
RT = 2000


def _mesh():
    return plsc.VectorSubcoreMesh(core_axis_name="c", subcore_axis_name="s")


def _deg_body(dst_hbm, zeros_hbm, ones_hbm, out_hbm, idx_v, ones_v, deg_s):
    cid = lax.axis_index("c")
    sid = lax.axis_index("s")
    gid = cid * NS + sid
    pltpu.sync_copy(zeros_hbm.at[pl.ds(sid * RPP, RPP)],
                    deg_s.at[pl.ds(sid * RPP, RPP)])
    pltpu.sync_copy(dst_hbm.at[gid], idx_v)
    pltpu.sync_copy(ones_hbm, ones_v)
    plsc.subcore_barrier()

    def body(j, c):
        pltpu.sync_copy(ones_v, deg_s.at[idx_v.at[j]], add=True)
        return c

    lax.fori_loop(0, NB, body, 0)
    plsc.subcore_barrier()
    pltpu.sync_copy(deg_s.at[pl.ds(sid * RPP, RPP)],
                    out_hbm.at[cid, pl.ds(sid * RPP, RPP)])


def _deg_call(dst3, zeros_n1, ones_k1):
    return pl.kernel(
        _deg_body,
        out_type=jax.ShapeDtypeStruct((NC, NP, 1), jnp.float32),
        mesh=_mesh(),
        scratch_types=[
            pltpu.VMEM((NB, K), jnp.int32),
            pltpu.VMEM((K, 1), jnp.float32),
            pltpu.VMEM_SHARED((NP, 1), jnp.float32),
        ],
    )(dst3, zeros_n1, ones_k1)


def _agg_body(hs_hbm, src_hbm, dst_hbm, zeros_hbm, out_hbm,
              src_v, dst_v, rows_v, acc_s, sem):
    cid = lax.axis_index("c")
    sid = lax.axis_index("s")
    gid = cid * NS + sid
    pltpu.sync_copy(zeros_hbm.at[pl.ds(sid * RP, RP)],
                    acc_s.at[pl.ds(sid * RP, RP)])
    pltpu.sync_copy(src_hbm.at[gid], src_v)
    pltpu.sync_copy(dst_hbm.at[gid], dst_v)
    plsc.subcore_barrier()

    def body(j, c):
        pltpu.async_copy(hs_hbm.at[src_v.at[j]], rows_v, sem).wait()
        pltpu.sync_copy(rows_v, acc_s.at[dst_v.at[j]], add=True)
        return c

    lax.fori_loop(0, NB, body, 0)
    plsc.subcore_barrier()
    pltpu.sync_copy(acc_s.at[pl.ds(sid * RP, RP)],
                    out_hbm.at[cid, pl.ds(sid * RP, RP)])


def _agg_call(hs, src3, dst3, zeros_nd):
    return pl.kernel(
        _agg_body,
        out_type=jax.ShapeDtypeStruct((NC, N, D), jnp.float32),
        mesh=_mesh(),
        scratch_types=[
            pltpu.VMEM((NB, K), jnp.int32),
            pltpu.VMEM((NB, K), jnp.int32),
            pltpu.VMEM((K, D), jnp.float32),
            pltpu.VMEM_SHARED((N, D), jnp.float32),
            pltpu.SemaphoreType.DMA,
        ],
    )(hs, src3, dst3, zeros_nd)


def _tc1_body(degp_ref, x_ref, w_ref, isd_ref, hs_ref):
    d = degp_ref[...]
    isd = lax.rsqrt(d[0] + d[1] + 1.0)
    isd_ref[...] = isd
    mm = jnp.dot(x_ref[...], w_ref[...], preferred_element_type=jnp.float32)
    hs_ref[...] = mm * isd


def _tc1(degp, x, W1):
    return pl.pallas_call(
        _tc1_body,
        grid=(N // RT,),
        in_specs=[
            pl.BlockSpec((NC, RT, 1), lambda i: (0, i, 0)),
            pl.BlockSpec((RT, D), lambda i: (i, 0)),
            pl.BlockSpec((D, D), lambda i: (0, 0)),
        ],
        out_specs=[
            pl.BlockSpec((RT, 1), lambda i: (i, 0)),
            pl.BlockSpec((RT, D), lambda i: (i, 0)),
        ],
        out_shape=[
            jax.ShapeDtypeStruct((N, 1), jnp.float32),
            jax.ShapeDtypeStruct((N, D), jnp.float32),
        ],
    )(degp, x, W1)


def _tc2_body(aggp_ref, hs_ref, isd_ref, w_ref, out_ref):
    a = aggp_ref[...]
    isd = isd_ref[...]
    h1 = jnp.maximum((a[0] + a[1] + hs_ref[...]) * isd, 0.0)
    mm = jnp.dot(h1, w_ref[...], preferred_element_type=jnp.float32)
    out_ref[...] = mm * isd


def _tc2(aggp, hs0, isd, W2):
    return pl.pallas_call(
        _tc2_body,
        grid=(N // RT,),
        in_specs=[
            pl.BlockSpec((NC, RT, D), lambda i: (0, i, 0)),
            pl.BlockSpec((RT, D), lambda i: (i, 0)),
            pl.BlockSpec((RT, 1), lambda i: (i, 0)),
            pl.BlockSpec((D, D), lambda i: (0, 0)),
        ],
        out_specs=pl.BlockSpec((RT, D), lambda i: (i, 0)),
        out_shape=jax.ShapeDtypeStruct((N, D), jnp.float32),
    )(aggp, hs0, isd, W2)


def _tc3_body(aggp_ref, hs_ref, isd_ref, out_ref):
    a = aggp_ref[...]
    out_ref[...] = (a[0] + a[1] + hs_ref[...]) * isd_ref[...]


def _tc3(aggp, hs1, isd):
    return pl.pallas_call(
        _tc3_body,
        grid=(N // RT,),
        in_specs=[
            pl.BlockSpec((NC, RT, D), lambda i: (0, i, 0)),
            pl.BlockSpec((RT, D), lambda i: (i, 0)),
            pl.BlockSpec((RT, 1), lambda i: (i, 0)),
        ],
        out_specs=pl.BlockSpec((RT, D), lambda i: (i, 0)),
        out_shape=jax.ShapeDtypeStruct((N, D), jnp.float32),
    )(aggp, hs1, isd)


def kernel(x, edge_index, W1, W2):
    edge_index = edge_index.astype(jnp.int32)
    src3 = edge_index[0].reshape(NW, NB, K)
    dst3 = edge_index[1].reshape(NW, NB, K)
    zeros_nd = jnp.zeros((N, D), jnp.float32)
    zeros_n1 = jnp.zeros((NP, 1), jnp.float32)
    ones_k1 = jnp.ones((K, 1), jnp.float32)

    degp = _deg_call(dst3, zeros_n1, ones_k1)
    isd, hs0 = _tc1(degp, x, W1)
    aggp1 = _agg_call(hs0, src3, dst3, zeros_nd)
    hs1 = _tc2(aggp1, hs0, isd, W2)
    aggp2 = _agg_call(hs1, src3, dst3, zeros_nd)
    return _tc3(aggp2, hs1, isd)


# speedup vs baseline: 15.5719x; 15.5719x over previous
import jax
import jax.numpy as jnp
from jax import lax
from jax.experimental import pallas as pl
from jax.experimental.pallas import tpu as pltpu
from jax.experimental.pallas import tpu_sc as plsc

N = 10000
D = 128
E = 320000
NC = 2
NS = 16
NW = NC * NS
PT = E // NW
K = 80
NB = PT // K
NP = 10240
RPP = NP // NS
RT = 2000


def _mesh():
    return plsc.VectorSubcoreMesh(core_axis_name="c", subcore_axis_name="s")


def _deg_body(dst_hbm, zeros_hbm, ones_hbm, out_hbm, idx_v, ones_v, deg_s):
    cid = lax.axis_index("c")
    sid = lax.axis_index("s")
    gid = cid * NS + sid
    pltpu.sync_copy(zeros_hbm.at[pl.ds(sid * RPP, RPP)],
                    deg_s.at[pl.ds(sid * RPP, RPP)])
    pltpu.sync_copy(dst_hbm.at[gid], idx_v)
    pltpu.sync_copy(ones_hbm, ones_v)
    plsc.subcore_barrier()

    def body(j, c):
        pltpu.sync_copy(ones_v, deg_s.at[idx_v.at[j]], add=True)
        return c

    lax.fori_loop(0, NB, body, 0)
    plsc.subcore_barrier()
    pltpu.sync_copy(deg_s.at[pl.ds(sid * RPP, RPP)],
                    out_hbm.at[cid, pl.ds(sid * RPP, RPP)])


def _deg_call(dst3, zeros_nd, ones_kd):
    return pl.kernel(
        _deg_body,
        out_type=jax.ShapeDtypeStruct((NC, NP, D), jnp.float32),
        mesh=_mesh(),
        scratch_types=[
            pltpu.VMEM((NB, K), jnp.int32),
            pltpu.VMEM((K, D), jnp.float32),
            pltpu.VMEM_SHARED((NP, D), jnp.float32),
        ],
    )(dst3, zeros_nd, ones_kd)


def _agg_body(hs_hbm, src_hbm, dst_hbm, zeros_hbm, out_hbm,
              src_v, dst_v, rows_v, acc_s, sem):
    cid = lax.axis_index("c")
    sid = lax.axis_index("s")
    gid = cid * NS + sid
    pltpu.sync_copy(zeros_hbm.at[pl.ds(sid * RPP, RPP)],
                    acc_s.at[pl.ds(sid * RPP, RPP)])
    pltpu.sync_copy(src_hbm.at[gid], src_v)
    pltpu.sync_copy(dst_hbm.at[gid], dst_v)
    plsc.subcore_barrier()

    def body(j, c):
        pltpu.async_copy(hs_hbm.at[src_v.at[j]], rows_v, sem).wait()
        pltpu.sync_copy(rows_v, acc_s.at[dst_v.at[j]], add=True)
        return c

    lax.fori_loop(0, NB, body, 0)
    plsc.subcore_barrier()
    pltpu.sync_copy(acc_s.at[pl.ds(sid * RPP, RPP)],
                    out_hbm.at[cid, pl.ds(sid * RPP, RPP)])


def _agg_call(hs, src3, dst3, zeros_nd):
    return pl.kernel(
        _agg_body,
        out_type=jax.ShapeDtypeStruct((NC, NP, D), jnp.float32),
        mesh=_mesh(),
        scratch_types=[
            pltpu.VMEM((NB, K), jnp.int32),
            pltpu.VMEM((NB, K), jnp.int32),
            pltpu.VMEM((K, D), jnp.float32),
            pltpu.VMEM_SHARED((NP, D), jnp.float32),
            pltpu.SemaphoreType.DMA,
        ],
    )(hs, src3, dst3, zeros_nd)


def _tc1_body(degp_ref, x_ref, w_ref, isd_ref, hs_ref):
    d = degp_ref[...]
    isd = lax.rsqrt(d[0] + d[1] + 1.0)
    isd_ref[...] = isd
    mm = jnp.dot(x_ref[...], w_ref[...], preferred_element_type=jnp.float32)
    hs_ref[...] = mm * isd


def _tc1(degp, x, W1):
    return pl.pallas_call(
        _tc1_body,
        grid=(N // RT,),
        in_specs=[
            pl.BlockSpec((NC, RT, D), lambda i: (0, i, 0)),
            pl.BlockSpec((RT, D), lambda i: (i, 0)),
            pl.BlockSpec((D, D), lambda i: (0, 0)),
        ],
        out_specs=[
            pl.BlockSpec((RT, D), lambda i: (i, 0)),
            pl.BlockSpec((RT, D), lambda i: (i, 0)),
        ],
        out_shape=[
            jax.ShapeDtypeStruct((N, D), jnp.float32),
            jax.ShapeDtypeStruct((N, D), jnp.float32),
        ],
    )(degp, x, W1)


def _tc2_body(aggp_ref, hs_ref, isd_ref, w_ref, out_ref):
    a = aggp_ref[...]
    isd = isd_ref[...]
    h1 = jnp.maximum((a[0] + a[1] + hs_ref[...]) * isd, 0.0)
    mm = jnp.dot(h1, w_ref[...], preferred_element_type=jnp.float32)
    out_ref[...] = mm * isd


def _tc2(aggp, hs0, isd, W2):
    return pl.pallas_call(
        _tc2_body,
        grid=(N // RT,),
        in_specs=[
            pl.BlockSpec((NC, RT, D), lambda i: (0, i, 0)),
            pl.BlockSpec((RT, D), lambda i: (i, 0)),
            pl.BlockSpec((RT, D), lambda i: (i, 0)),
            pl.BlockSpec((D, D), lambda i: (0, 0)),
        ],
        out_specs=pl.BlockSpec((RT, D), lambda i: (i, 0)),
        out_shape=jax.ShapeDtypeStruct((N, D), jnp.float32),
    )(aggp, hs0, isd, W2)


def _tc3_body(aggp_ref, hs_ref, isd_ref, out_ref):
    a = aggp_ref[...]
    out_ref[...] = (a[0] + a[1] + hs_ref[...]) * isd_ref[...]


def _tc3(aggp, hs1, isd):
    return pl.pallas_call(
        _tc3_body,
        grid=(N // RT,),
        in_specs=[
            pl.BlockSpec((NC, RT, D), lambda i: (0, i, 0)),
            pl.BlockSpec((RT, D), lambda i: (i, 0)),
            pl.BlockSpec((RT, D), lambda i: (i, 0)),
        ],
        out_specs=pl.BlockSpec((RT, D), lambda i: (i, 0)),
        out_shape=jax.ShapeDtypeStruct((N, D), jnp.float32),
    )(aggp, hs1, isd)


def kernel(x, edge_index, W1, W2):
    edge_index = edge_index.astype(jnp.int32)
    src3 = edge_index[0].reshape(NW, NB, K)
    dst3 = edge_index[1].reshape(NW, NB, K)
    zeros_nd = jnp.zeros((NP, D), jnp.float32)
    ones_kd = jnp.ones((K, D), jnp.float32)

    degp = _deg_call(dst3, zeros_nd, ones_kd)
    isd, hs0 = _tc1(degp, x, W1)
    aggp1 = _agg_call(hs0, src3, dst3, zeros_nd)
    hs1 = _tc2(aggp1, hs0, isd, W2)
    aggp2 = _agg_call(hs1, src3, dst3, zeros_nd)
    return _tc3(aggp2, hs1, isd)


# speedup vs baseline: 22.8244x; 1.4657x over previous
import jax
import jax.numpy as jnp
from jax import lax
from jax.experimental import pallas as pl
from jax.experimental.pallas import tpu as pltpu
from jax.experimental.pallas import tpu_sc as plsc

N = 10000
D = 128
E = 320000
NC = 2
NS = 16
NW = NC * NS
K = 128
PT = 10240
EP = NW * PT
NB = PT // K
HB = NB // 2
NT = HB // 2
NTD = NB // 2
NP = 10240
RPP = NP // NS
RT = 2048


def _mesh():
    return plsc.VectorSubcoreMesh(core_axis_name="c", subcore_axis_name="s")


def _deg_body(dst_hbm, out_hbm, idx_v, ones_v, zb_v, deg_s, sem0, sem1):
    cid = lax.axis_index("c")
    sid = lax.axis_index("s")
    gid = cid * NS + sid

    def zbody(i, c):
        zb_v[pl.ds(i * 16, 16)] = jnp.zeros((16,), jnp.float32)
        return c

    lax.fori_loop(0, RPP // 16, zbody, 0)

    def obody(i, c):
        ones_v[pl.ds(i * 16, 16)] = jnp.ones((16,), jnp.float32)
        return c

    lax.fori_loop(0, K // 16, obody, 0)
    pltpu.sync_copy(zb_v, deg_s.at[pl.ds(sid * RPP, RPP)])
    pltpu.sync_copy(dst_hbm.at[gid], idx_v)
    plsc.subcore_barrier()

    def body(t, c):
        pltpu.async_copy(ones_v, deg_s.at[idx_v.at[2 * t]], sem0, add=True)
        pltpu.async_copy(ones_v, deg_s.at[idx_v.at[2 * t + 1]], sem1, add=True)
        pltpu.make_async_copy(ones_v, deg_s.at[idx_v.at[2 * t]], sem0).wait()
        pltpu.make_async_copy(ones_v, deg_s.at[idx_v.at[2 * t + 1]],
                              sem1).wait()
        return c

    lax.fori_loop(0, NTD, body, 0)
    plsc.subcore_barrier()
    pltpu.sync_copy(deg_s.at[pl.ds(sid * RPP, RPP)],
                    out_hbm.at[pl.ds(cid * NP + sid * RPP, RPP)])


def _deg_call(dst3):
    return pl.kernel(
        _deg_body,
        out_type=jax.ShapeDtypeStruct((NC * NP,), jnp.float32),
        mesh=_mesh(),
        scratch_types=[
            pltpu.VMEM((NB, K), jnp.int32),
            pltpu.VMEM((K,), jnp.float32),
            pltpu.VMEM((RPP,), jnp.float32),
            pltpu.VMEM_SHARED((NP,), jnp.float32),
            pltpu.SemaphoreType.DMA,
            pltpu.SemaphoreType.DMA,
        ],
    )(dst3)


def _agg_body(hs_hbm, src_hbm, dst_hbm, zeros_hbm, out_hbm,
              src_v, dst_v, rows_v, acc_s, gsem0, gsem1, ssem0, ssem1):
    cid = lax.axis_index("c")
    sid = lax.axis_index("s")
    gid = cid * NS + sid
    pltpu.sync_copy(zeros_hbm.at[pl.ds(sid * RPP, RPP)],
                    acc_s.at[pl.ds(sid * RPP, RPP)])

    for p in range(2):
        pltpu.sync_copy(src_hbm.at[gid, pl.ds(p * HB, HB)], src_v)
        pltpu.sync_copy(dst_hbm.at[gid, pl.ds(p * HB, HB)], dst_v)

        pltpu.async_copy(hs_hbm.at[src_v.at[0]], rows_v.at[0], gsem0)
        pltpu.async_copy(hs_hbm.at[src_v.at[1]], rows_v.at[1], gsem1)

        if p == 0:
            plsc.subcore_barrier()

        def body(t, c):
            j0 = 2 * t
            j1 = 2 * t + 1
            pltpu.make_async_copy(hs_hbm.at[src_v.at[j0]], rows_v.at[0],
                                  gsem0).wait()
            pltpu.async_copy(rows_v.at[0], acc_s.at[dst_v.at[j0]], ssem0,
                             add=True)
            pltpu.make_async_copy(hs_hbm.at[src_v.at[j1]], rows_v.at[1],
                                  gsem1).wait()
            pltpu.async_copy(rows_v.at[1], acc_s.at[dst_v.at[j1]], ssem1,
                             add=True)
            pltpu.make_async_copy(rows_v.at[0], acc_s.at[dst_v.at[j0]],
                                  ssem0).wait()

            @pl.when(t + 1 < NT)
            def _():
                pltpu.async_copy(hs_hbm.at[src_v.at[j0 + 2]], rows_v.at[0],
                                 gsem0)

            pltpu.make_async_copy(rows_v.at[1], acc_s.at[dst_v.at[j1]],
                                  ssem1).wait()

            @pl.when(t + 1 < NT)
            def _():
                pltpu.async_copy(hs_hbm.at[src_v.at[j1 + 2]], rows_v.at[1],
                                 gsem1)

            return c

        lax.fori_loop(0, NT, body, 0)

    plsc.subcore_barrier()
    pltpu.sync_copy(acc_s.at[pl.ds(sid * RPP, RPP)],
                    out_hbm.at[cid, pl.ds(sid * RPP, RPP)])


def _agg_call(hs, src3, dst3, zeros_nd):
    return pl.kernel(
        _agg_body,
        out_type=jax.ShapeDtypeStruct((NC, NP, D), jnp.float32),
        mesh=_mesh(),
        scratch_types=[
            pltpu.VMEM((HB, K), jnp.int32),
            pltpu.VMEM((HB, K), jnp.int32),
            pltpu.VMEM((2, K, D), jnp.float32),
            pltpu.VMEM_SHARED((NP, D), jnp.float32),
            pltpu.SemaphoreType.DMA,
            pltpu.SemaphoreType.DMA,
            pltpu.SemaphoreType.DMA,
            pltpu.SemaphoreType.DMA,
        ],
    )(hs, src3, dst3, zeros_nd)


def _tc0_body(degp_ref, isd_ref):
    d = degp_ref[...]
    isd_ref[...] = lax.rsqrt(d[0] + d[1] + 1.0)


def _tc0(degp3):
    return pl.pallas_call(
        _tc0_body,
        out_shape=jax.ShapeDtypeStruct((NP // 128, 128), jnp.float32),
    )(degp3)


def _tc1_body(x_ref, w_ref, isd_ref, hs_ref):
    mm = jnp.dot(x_ref[...], w_ref[...], preferred_element_type=jnp.float32)
    hs_ref[...] = mm * isd_ref[...]


def _tc1(x_p, W1, isd_col):
    return pl.pallas_call(
        _tc1_body,
        grid=(NP // RT,),
        in_specs=[
            pl.BlockSpec((RT, D), lambda i: (i, 0)),
            pl.BlockSpec((D, D), lambda i: (0, 0)),
            pl.BlockSpec((RT, 1), lambda i: (i, 0)),
        ],
        out_specs=pl.BlockSpec((RT, D), lambda i: (i, 0)),
        out_shape=jax.ShapeDtypeStruct((NP, D), jnp.float32),
    )(x_p, W1, isd_col)


def _tc2_body(aggp_ref, hs_ref, isd_ref, w_ref, out_ref):
    a = aggp_ref[...]
    isd = isd_ref[...]
    h1 = jnp.maximum((a[0] + a[1] + hs_ref[...]) * isd, 0.0)
    mm = jnp.dot(h1, w_ref[...], preferred_element_type=jnp.float32)
    out_ref[...] = mm * isd


def _tc2(aggp, hs0, isd_col, W2):
    return pl.pallas_call(
        _tc2_body,
        grid=(NP // RT,),
        in_specs=[
            pl.BlockSpec((NC, RT, D), lambda i: (0, i, 0)),
            pl.BlockSpec((RT, D), lambda i: (i, 0)),
            pl.BlockSpec((RT, 1), lambda i: (i, 0)),
            pl.BlockSpec((D, D), lambda i: (0, 0)),
        ],
        out_specs=pl.BlockSpec((RT, D), lambda i: (i, 0)),
        out_shape=jax.ShapeDtypeStruct((NP, D), jnp.float32),
    )(aggp, hs0, isd_col, W2)


def _tc3_body(aggp_ref, hs_ref, isd_ref, out_ref):
    a = aggp_ref[...]
    out_ref[...] = (a[0] + a[1] + hs_ref[...]) * isd_ref[...]


def _tc3(aggp, hs1, isd_col):
    return pl.pallas_call(
        _tc3_body,
        grid=(NP // RT,),
        in_specs=[
            pl.BlockSpec((NC, RT, D), lambda i: (0, i, 0)),
            pl.BlockSpec((RT, D), lambda i: (i, 0)),
            pl.BlockSpec((RT, 1), lambda i: (i, 0)),
        ],
        out_specs=pl.BlockSpec((RT, D), lambda i: (i, 0)),
        out_shape=jax.ShapeDtypeStruct((NP, D), jnp.float32),
    )(aggp, hs1, isd_col)


def kernel(x, edge_index, W1, W2):
    edge_index = edge_index.astype(jnp.int32)
    npad = EP - E
    pad_src = jnp.arange(npad, dtype=jnp.int32) % N
    pad_dst = N + (jnp.arange(npad, dtype=jnp.int32) % (NP - N))
    src3 = jnp.concatenate([edge_index[0], pad_src]).reshape(NW, NB, K)
    dst3 = jnp.concatenate([edge_index[1], pad_dst]).reshape(NW, NB, K)
    zeros_nd = jnp.zeros((NP, D), jnp.float32)
    x_p = jnp.pad(x, ((0, NP - N), (0, 0)))

    degf = _deg_call(dst3)
    isd_col = _tc0(degf.reshape(NC, NP // 128, 128)).reshape(NP, 1)
    hs0 = _tc1(x_p, W1, isd_col)
    aggp1 = _agg_call(hs0, src3, dst3, zeros_nd)
    hs1 = _tc2(aggp1, hs0, isd_col, W2)
    aggp2 = _agg_call(hs1, src3, dst3, zeros_nd)
    return _tc3(aggp2, hs1, isd_col)[:N]
